# Initial kernel scaffold; baseline (speedup 1.0000x reference)
import jax
import jax.numpy as jnp
from jax.experimental import pallas as pl


def kernel(boxes, scores):
    raise NotImplementedError("write your pallas kernel here")


# speedup vs baseline: 5.0833x; 5.0833x over previous
import functools

import jax
import jax.numpy as jnp
from jax import lax
from jax.experimental import pallas as pl
from jax.experimental.pallas import tpu as pltpu
from jax.experimental.pallas import tpu_sc as plsc

_GATHER_DNUMS = lax.GatherDimensionNumbers(
    offset_dims=(), collapsed_slice_dims=(0,), start_index_map=(0,))


def _permute(v, idx):
    return lax.gather(v, idx[:, None], _GATHER_DNUMS, (1,),
                      mode=lax.GatherScatterMode.PROMISE_IN_BOUNDS)


def _butterfly(v, op, lane):
    for sh in (1, 2, 4, 8):
        v = op(v, _permute(v, lane ^ sh))
    return v


_N = 20000
_MAX_DET = 40
_IOU_THR = 0.5
_NEG = -1e30

_LANES = 16
_TILES = 16
_PER_TILE = 1280
_NV = _PER_TILE // _LANES
_NPAD = _TILES * _PER_TILE


def _nms_body(x1_hbm, y1_hbm, x2_hbm, y2_hbm, sc_hbm, out_hbm,
              X1, Y1, X2, Y2, AREA, SCOR, WORK, REC, BLK, OUT, SHARED):
    c = lax.axis_index("c")
    s = lax.axis_index("s")
    base = s * _PER_TILE

    pltpu.sync_copy(x1_hbm.at[pl.ds(base, _PER_TILE)], X1)
    pltpu.sync_copy(y1_hbm.at[pl.ds(base, _PER_TILE)], Y1)
    pltpu.sync_copy(x2_hbm.at[pl.ds(base, _PER_TILE)], X2)
    pltpu.sync_copy(y2_hbm.at[pl.ds(base, _PER_TILE)], Y2)
    pltpu.sync_copy(sc_hbm.at[pl.ds(base, _PER_TILE)], SCOR)
    pltpu.sync_copy(sc_hbm.at[pl.ds(base, _PER_TILE)], WORK)

    lane = lax.iota(jnp.int32, _LANES)
    neg16 = jnp.full((_LANES,), _NEG, jnp.float32)
    zero16i = jnp.zeros((_LANES,), jnp.int32)

    def publish(bestv, bestj):
        m = _butterfly(bestv, jnp.maximum, lane)
        wlane = _butterfly(jnp.where(bestv == m, lane, _LANES),
                           jnp.minimum, lane)
        jloc = _butterfly(jnp.where(lane == wlane, bestj, 0),
                          jnp.maximum, lane)
        liv = jloc * _LANES + wlane
        wx1 = plsc.load_gather(X1, [liv])
        wy1 = plsc.load_gather(Y1, [liv])
        wx2 = plsc.load_gather(X2, [liv])
        wy2 = plsc.load_gather(Y2, [liv])
        wa = plsc.load_gather(AREA, [liv])
        wos = plsc.load_gather(SCOR, [liv])
        rec = jnp.where(lane == 0, m,
              jnp.where(lane == 1, wx1,
              jnp.where(lane == 2, wy1,
              jnp.where(lane == 3, wx2,
              jnp.where(lane == 4, wy2,
              jnp.where(lane == 5, wa,
              jnp.where(lane == 6, wos, 0.0)))))))
        REC[...] = rec
        pltpu.sync_copy(REC, SHARED.at[pl.ds(s * _LANES, _LANES)])

    def init_body(j, carry):
        bestv, bestj = carry
        sl = pl.ds(j * _LANES, _LANES)
        area = (jnp.maximum(X2[sl] - X1[sl], 0.0) *
                jnp.maximum(Y2[sl] - Y1[sl], 0.0))
        AREA[sl] = area
        w = WORK[sl]
        upd = w > bestv
        return jnp.where(upd, w, bestv), jnp.where(upd, j, bestj)

    bestv, bestj = lax.fori_loop(0, _NV, init_body, (neg16, zero16i))
    publish(bestv, bestj)
    plsc.subcore_barrier()

    def round_body(i, _):
        pltpu.sync_copy(SHARED, BLK)
        cscore = plsc.load_gather(BLK, [lane * _LANES])
        m = _butterfly(cscore, jnp.maximum, lane)
        wtv = _butterfly(jnp.where(cscore == m, lane, _LANES),
                         jnp.minimum, lane)
        wbase = wtv * _LANES
        wx1 = plsc.load_gather(BLK, [wbase + 1])
        wy1 = plsc.load_gather(BLK, [wbase + 2])
        wx2 = plsc.load_gather(BLK, [wbase + 3])
        wy2 = plsc.load_gather(BLK, [wbase + 4])
        wa = plsc.load_gather(BLK, [wbase + 5])
        wos = plsc.load_gather(BLK, [wbase + 6])

        det = jnp.where(lane == 0, wx1,
              jnp.where(lane == 1, wy1,
              jnp.where(lane == 2, wx2,
              jnp.where(lane == 3, wy2,
              jnp.where(lane == 4, wos, 0.0)))))
        plsc.store_scatter(OUT, [i * _LANES + lane], det)

        def sup_body(j, carry):
            bestv, bestj = carry
            sl = pl.ds(j * _LANES, _LANES)
            ix1 = jnp.maximum(X1[sl], wx1)
            iy1 = jnp.maximum(Y1[sl], wy1)
            ix2 = jnp.minimum(X2[sl], wx2)
            iy2 = jnp.minimum(Y2[sl], wy2)
            inter = (jnp.maximum(ix2 - ix1, 0.0) *
                     jnp.maximum(iy2 - iy1, 0.0))
            iou = inter / (wa + AREA[sl] - inter + 1e-8)
            w = jnp.where(iou > _IOU_THR, _NEG, WORK[sl])
            WORK[sl] = w
            upd = w > bestv
            return jnp.where(upd, w, bestv), jnp.where(upd, j, bestj)

        bestv, bestj = lax.fori_loop(0, _NV, sup_body, (neg16, zero16i))
        plsc.subcore_barrier()
        publish(bestv, bestj)
        plsc.subcore_barrier()
        return 0

    lax.fori_loop(0, _MAX_DET, round_body, 0)

    @pl.when((c == 0) & (s == 0))
    def _():
        pltpu.sync_copy(OUT, out_hbm)


@jax.jit
def _nms_sc(x1, y1, x2, y2, sc):
    mesh = plsc.VectorSubcoreMesh(core_axis_name="c", subcore_axis_name="s",
                                  num_cores=2, num_subcores=16)
    f = functools.partial(
        pl.kernel,
        out_type=jax.ShapeDtypeStruct((_MAX_DET * _LANES,), jnp.float32),
        mesh=mesh,
        compiler_params=pltpu.CompilerParams(needs_layout_passes=False),
        scratch_types=[
            pltpu.VMEM((_PER_TILE,), jnp.float32),
            pltpu.VMEM((_PER_TILE,), jnp.float32),
            pltpu.VMEM((_PER_TILE,), jnp.float32),
            pltpu.VMEM((_PER_TILE,), jnp.float32),
            pltpu.VMEM((_PER_TILE,), jnp.float32),
            pltpu.VMEM((_PER_TILE,), jnp.float32),
            pltpu.VMEM((_PER_TILE,), jnp.float32),
            pltpu.VMEM((_LANES,), jnp.float32),
            pltpu.VMEM((_TILES * _LANES,), jnp.float32),
            pltpu.VMEM((_MAX_DET * _LANES,), jnp.float32),
            pltpu.VMEM_SHARED((_TILES * _LANES,), jnp.float32),
        ],
    )(_nms_body)
    return f(x1, y1, x2, y2, sc)


def kernel(boxes, scores):
    x1 = jnp.zeros((_NPAD,), jnp.float32).at[:_N].set(boxes[:, 0])
    y1 = jnp.zeros((_NPAD,), jnp.float32).at[:_N].set(boxes[:, 1])
    x2 = jnp.zeros((_NPAD,), jnp.float32).at[:_N].set(boxes[:, 2])
    y2 = jnp.zeros((_NPAD,), jnp.float32).at[:_N].set(boxes[:, 3])
    sc = jnp.full((_NPAD,), _NEG, jnp.float32).at[:_N].set(scores)
    flat = _nms_sc(x1, y1, x2, y2, sc)
    return flat.reshape(_MAX_DET, _LANES)[:, :5]


# speedup vs baseline: 10.1607x; 1.9989x over previous
import functools

import jax
import jax.numpy as jnp
from jax import lax
from jax.experimental import pallas as pl
from jax.experimental.pallas import tpu as pltpu
from jax.experimental.pallas import tpu_sc as plsc

_GATHER_DNUMS = lax.GatherDimensionNumbers(
    offset_dims=(), collapsed_slice_dims=(0,), start_index_map=(0,))


def _permute(v, idx):
    return lax.gather(v, idx[:, None], _GATHER_DNUMS, (1,),
                      mode=lax.GatherScatterMode.PROMISE_IN_BOUNDS)


def _butterfly(v, op, lane):
    for sh in (1, 2, 4, 8):
        v = op(v, _permute(v, lane ^ sh))
    return v


_N = 20000
_MAX_DET = 40
_IOU_THR = 0.5
_NEG = -1e30

_LANES = 16
_TILES = 16
_PER_TILE = 1280
_NV = _PER_TILE // _LANES
_NPAD = _TILES * _PER_TILE
_BLKW = _TILES * _LANES


def _nms_body(x1_hbm, y1_hbm, x2_hbm, y2_hbm, sc_hbm, out_hbm,
              X1, Y1, X2, Y2, AREA, SCOR, WORK, REC, BLK, OUT, SHARED):
    c = lax.axis_index("c")
    s = lax.axis_index("s")
    base = s * _PER_TILE

    pltpu.sync_copy(x1_hbm.at[pl.ds(base, _PER_TILE)], X1)
    pltpu.sync_copy(y1_hbm.at[pl.ds(base, _PER_TILE)], Y1)
    pltpu.sync_copy(x2_hbm.at[pl.ds(base, _PER_TILE)], X2)
    pltpu.sync_copy(y2_hbm.at[pl.ds(base, _PER_TILE)], Y2)
    pltpu.sync_copy(sc_hbm.at[pl.ds(base, _PER_TILE)], SCOR)
    pltpu.sync_copy(sc_hbm.at[pl.ds(base, _PER_TILE)], WORK)

    lane = lax.iota(jnp.int32, _LANES)
    neg16 = jnp.full((_LANES,), _NEG, jnp.float32)
    zero16i = jnp.zeros((_LANES,), jnp.int32)

    def publish(bestv, bestj, off):
        m = _butterfly(bestv, jnp.maximum, lane)
        wlane = _butterfly(jnp.where(bestv == m, lane, _LANES),
                           jnp.minimum, lane)
        jloc = _butterfly(jnp.where(lane == wlane, bestj, 0),
                          jnp.maximum, lane)
        liv = jloc * _LANES + wlane
        wx1 = plsc.load_gather(X1, [liv])
        wy1 = plsc.load_gather(Y1, [liv])
        wx2 = plsc.load_gather(X2, [liv])
        wy2 = plsc.load_gather(Y2, [liv])
        wa = plsc.load_gather(AREA, [liv])
        wos = plsc.load_gather(SCOR, [liv])
        rec = jnp.where(lane == 0, m,
              jnp.where(lane == 1, wx1,
              jnp.where(lane == 2, wy1,
              jnp.where(lane == 3, wx2,
              jnp.where(lane == 4, wy2,
              jnp.where(lane == 5, wa,
              jnp.where(lane == 6, wos, 0.0)))))))
        REC[...] = rec
        pltpu.sync_copy(REC, SHARED.at[pl.ds(off + s * _LANES, _LANES)])

    @plsc.parallel_loop(0, _NV, carry=(neg16, zero16i), unroll=4)
    def init_loop(j, carry):
        bestv, bestj = carry
        sl = pl.ds(j * _LANES, _LANES)
        area = (jnp.maximum(X2[sl] - X1[sl], 0.0) *
                jnp.maximum(Y2[sl] - Y1[sl], 0.0))
        AREA[sl] = area
        w = WORK[sl]
        upd = w > bestv
        return jnp.where(upd, w, bestv), jnp.where(upd, j, bestj)

    bestv, bestj = init_loop
    publish(bestv, bestj, 0)
    plsc.subcore_barrier()

    def round_body(i, _):
        pr = jnp.bitwise_and(i, 1)
        pltpu.sync_copy(SHARED.at[pl.ds(pr * _BLKW, _BLKW)], BLK)
        cscore = plsc.load_gather(BLK, [lane * _LANES])
        m = _butterfly(cscore, jnp.maximum, lane)
        wtv = _butterfly(jnp.where(cscore == m, lane, _LANES),
                         jnp.minimum, lane)
        wbase = wtv * _LANES
        wx1 = plsc.load_gather(BLK, [wbase + 1])
        wy1 = plsc.load_gather(BLK, [wbase + 2])
        wx2 = plsc.load_gather(BLK, [wbase + 3])
        wy2 = plsc.load_gather(BLK, [wbase + 4])
        wa = plsc.load_gather(BLK, [wbase + 5])
        wos = plsc.load_gather(BLK, [wbase + 6])

        det = jnp.where(lane == 0, wx1,
              jnp.where(lane == 1, wy1,
              jnp.where(lane == 2, wx2,
              jnp.where(lane == 3, wy2,
              jnp.where(lane == 4, wos, 0.0)))))
        plsc.store_scatter(OUT, [i * _LANES + lane], det)

        @plsc.parallel_loop(0, _NV, carry=(neg16, zero16i), unroll=4)
        def sup_loop(j, carry):
            bestv, bestj = carry
            sl = pl.ds(j * _LANES, _LANES)
            ix1 = jnp.maximum(X1[sl], wx1)
            iy1 = jnp.maximum(Y1[sl], wy1)
            ix2 = jnp.minimum(X2[sl], wx2)
            iy2 = jnp.minimum(Y2[sl], wy2)
            inter = (jnp.maximum(ix2 - ix1, 0.0) *
                     jnp.maximum(iy2 - iy1, 0.0))
            denom = wa + AREA[sl] - inter + 1e-8
            w = jnp.where(inter > _IOU_THR * denom, _NEG, WORK[sl])
            WORK[sl] = w
            upd = w > bestv
            return jnp.where(upd, w, bestv), jnp.where(upd, j, bestj)

        bestv, bestj = sup_loop
        publish(bestv, bestj, (1 - pr) * _BLKW)
        plsc.subcore_barrier()
        return 0

    lax.fori_loop(0, _MAX_DET, round_body, 0)

    @pl.when((c == 0) & (s == 0))
    def _():
        pltpu.sync_copy(OUT, out_hbm)


@jax.jit
def _nms_sc(x1, y1, x2, y2, sc):
    mesh = plsc.VectorSubcoreMesh(core_axis_name="c", subcore_axis_name="s",
                                  num_cores=2, num_subcores=16)
    f = functools.partial(
        pl.kernel,
        out_type=jax.ShapeDtypeStruct((_MAX_DET * _LANES,), jnp.float32),
        mesh=mesh,
        compiler_params=pltpu.CompilerParams(needs_layout_passes=False),
        scratch_types=[
            pltpu.VMEM((_PER_TILE,), jnp.float32),
            pltpu.VMEM((_PER_TILE,), jnp.float32),
            pltpu.VMEM((_PER_TILE,), jnp.float32),
            pltpu.VMEM((_PER_TILE,), jnp.float32),
            pltpu.VMEM((_PER_TILE,), jnp.float32),
            pltpu.VMEM((_PER_TILE,), jnp.float32),
            pltpu.VMEM((_PER_TILE,), jnp.float32),
            pltpu.VMEM((_LANES,), jnp.float32),
            pltpu.VMEM((_BLKW,), jnp.float32),
            pltpu.VMEM((_MAX_DET * _LANES,), jnp.float32),
            pltpu.VMEM_SHARED((2 * _BLKW,), jnp.float32),
        ],
    )(_nms_body)
    return f(x1, y1, x2, y2, sc)


def kernel(boxes, scores):
    x1 = jnp.zeros((_NPAD,), jnp.float32).at[:_N].set(boxes[:, 0])
    y1 = jnp.zeros((_NPAD,), jnp.float32).at[:_N].set(boxes[:, 1])
    x2 = jnp.zeros((_NPAD,), jnp.float32).at[:_N].set(boxes[:, 2])
    y2 = jnp.zeros((_NPAD,), jnp.float32).at[:_N].set(boxes[:, 3])
    sc = jnp.full((_NPAD,), _NEG, jnp.float32).at[:_N].set(scores)
    flat = _nms_sc(x1, y1, x2, y2, sc)
    return flat.reshape(_MAX_DET, _LANES)[:, :5]
